# baseline (speedup 1.0000x reference)
import jax
import jax.numpy as jnp
from jax import lax
from jax.experimental import pallas as pl
from jax.experimental.pallas import tpu as pltpu
from jax.experimental.pallas import tpu_sc as plsc

BATCH = 16384
FACTORS = 64
TABLE_N = 1000000
ROW2 = 2 * FACTORS

_info = plsc.get_sparse_core_info()
_NC, _NS, _L = _info.num_cores, _info.num_subcores, _info.num_lanes
_NW = _NC * _NS
_BPW = BATCH // _NW
_GROUPS = _BPW // _L
_CH = 128
_NCH = _BPW // _CH
_CG = _CH // _L

_MESH = plsc.VectorSubcoreMesh(core_axis_name="c", subcore_axis_name="s")
_PARAMS = pltpu.CompilerParams(
    needs_layout_passes=False, use_tc_tiling_on_sc=True,
    skip_device_barrier=True)


def _gather_body(idx_hbm, tab_hbm, rows_hbm, idx_v, jidx0, jidx1,
                 rbuf0, rbuf1, sem0, sem1):
    wid = lax.axis_index("s") * _NC + lax.axis_index("c")
    base = wid * _BPW
    pltpu.sync_copy(idx_hbm.at[pl.ds(base, _BPW)], idx_v)

    jidx = (jidx0, jidx1)
    rbuf = (rbuf0, rbuf1)
    sems = (sem0, sem1)

    def fire(c, s):
        def body(g, carry):
            jidx[s][pl.ds(g * _L, _L)] = \
                idx_v[pl.ds(c * _CH + g * _L, _L)] >> 1
            return carry
        lax.fori_loop(0, _CG, body, 0)
        pltpu.async_copy(tab_hbm.at[jidx[s]], rbuf[s], sems[s])

    def drain(s):
        pltpu.make_async_copy(tab_hbm.at[jidx[s]], rbuf[s], sems[s]).wait()

    fire(0, 0)
    fire(1, 1)
    for c in range(_NCH):
        s = c % 2
        drain(s)
        pltpu.sync_copy(rbuf[s],
                        rows_hbm.at[pl.ds(base + c * _CH, _CH), :])
        if c + 2 < _NCH:
            fire(c + 2, s)


def _combine_body(user_hbm, item_hbm, urows_hbm, irows_hbm, ub_hbm, ib_hbm,
                  b_hbm, out_hbm, uidx_v, iidx_v, ubuf0, ubuf1, ibuf0,
                  ibuf1, ub_v, ib_v, bias_v, out_v, sem0, sem1, semb):
    wid = lax.axis_index("s") * _NC + lax.axis_index("c")
    base = wid * _BPW

    pltpu.sync_copy(user_hbm.at[pl.ds(base, _BPW)], uidx_v)
    pltpu.sync_copy(item_hbm.at[pl.ds(base, _BPW)], iidx_v)
    pltpu.sync_copy(b_hbm, bias_v)

    cp_ub = pltpu.async_copy(ub_hbm.at[uidx_v], ub_v, semb)
    cp_ib = pltpu.async_copy(ib_hbm.at[iidx_v], ib_v, semb)

    ubuf = (ubuf0, ubuf1)
    ibuf = (ibuf0, ibuf1)
    sems = (sem0, sem1)

    def fire(c, s):
        sl = pl.ds(base + c * _CH, _CH)
        pltpu.async_copy(urows_hbm.at[sl, :], ubuf[s], sems[s])
        pltpu.async_copy(irows_hbm.at[sl, :], ibuf[s], sems[s])

    def drain(c, s):
        sl = pl.ds(base + c * _CH, _CH)
        pltpu.make_async_copy(urows_hbm.at[sl, :], ubuf[s], sems[s]).wait()
        pltpu.make_async_copy(irows_hbm.at[sl, :], ibuf[s], sems[s]).wait()

    fire(0, 0)
    fire(1, 1)

    cp_ub.wait()
    cp_ib.wait()
    bias = bias_v[...]

    def seed(g, carry):
        sl = pl.ds(g * _L, _L)
        out_v[sl] = bias + ub_v[sl] + ib_v[sl]
        return carry
    lax.fori_loop(0, _GROUPS, seed, 0)

    lanes = lax.iota(jnp.int32, _L)

    def extract(c, s):
        def body(g, carry):
            sl = pl.ds(c * _CH + g * _L, _L)
            row = g * _L + lanes
            uhalf = (uidx_v[sl] & 1) * FACTORS
            ihalf = (iidx_v[sl] & 1) * FACTORS
            acc = out_v[sl]
            for f in range(FACTORS):
                u = plsc.load_gather(ubuf[s], [row, uhalf + f])
                v = plsc.load_gather(ibuf[s], [row, ihalf + f])
                acc = acc + u * v
            out_v[sl] = acc
            return carry
        lax.fori_loop(0, _CG, body, 0)

    for c in range(_NCH):
        s = c % 2
        drain(c, s)
        extract(c, s)
        if c + 2 < _NCH:
            fire(c + 2, s)

    pltpu.sync_copy(out_v, out_hbm.at[pl.ds(base, _BPW)])


def kernel(user, item, embed_user, embed_item, u_bias, i_bias, bias_):
    gather = pl.kernel(
        _gather_body,
        out_type=jax.ShapeDtypeStruct((BATCH, ROW2), jnp.float32),
        mesh=_MESH,
        compiler_params=_PARAMS,
        scratch_types=[
            pltpu.VMEM((_BPW,), jnp.int32),
            pltpu.VMEM((_CH,), jnp.int32),
            pltpu.VMEM((_CH,), jnp.int32),
            pltpu.VMEM((_CH, ROW2), jnp.float32),
            pltpu.VMEM((_CH, ROW2), jnp.float32),
            pltpu.SemaphoreType.DMA,
            pltpu.SemaphoreType.DMA,
        ],
    )
    combine = pl.kernel(
        _combine_body,
        out_type=jax.ShapeDtypeStruct((BATCH,), jnp.float32),
        mesh=_MESH,
        compiler_params=_PARAMS,
        scratch_types=[
            pltpu.VMEM((_BPW,), jnp.int32),
            pltpu.VMEM((_BPW,), jnp.int32),
            pltpu.VMEM((_CH, ROW2), jnp.float32),
            pltpu.VMEM((_CH, ROW2), jnp.float32),
            pltpu.VMEM((_CH, ROW2), jnp.float32),
            pltpu.VMEM((_CH, ROW2), jnp.float32),
            pltpu.VMEM((_BPW,), jnp.float32),
            pltpu.VMEM((_BPW,), jnp.float32),
            pltpu.VMEM((_L,), jnp.float32),
            pltpu.VMEM((_BPW,), jnp.float32),
            pltpu.SemaphoreType.DMA,
            pltpu.SemaphoreType.DMA,
            pltpu.SemaphoreType.DMA,
        ],
    )
    user = user.astype(jnp.int32)
    item = item.astype(jnp.int32)
    urows = gather(user, embed_user.reshape(TABLE_N // 2, ROW2))
    irows = gather(item, embed_item.reshape(TABLE_N // 2, ROW2))
    return combine(user, item, urows, irows,
                   u_bias.reshape(-1), i_bias.reshape(-1),
                   jnp.broadcast_to(bias_, (_L,)))


# speedup vs baseline: 1.0832x; 1.0832x over previous
import jax
import jax.numpy as jnp
from jax import lax
from jax.experimental import pallas as pl
from jax.experimental.pallas import tpu as pltpu
from jax.experimental.pallas import tpu_sc as plsc

BATCH = 16384
FACTORS = 64

_info = plsc.get_sparse_core_info()
_NC, _NS, _L = _info.num_cores, _info.num_subcores, _info.num_lanes
_NW = _NC * _NS
_BPW = BATCH // _NW
_GROUPS = _BPW // _L


def _fm_body(user_hbm, item_hbm, eu_hbm, ei_hbm, ub_hbm, ib_hbm, b_hbm,
             out_hbm, uidx_v, iidx_v, urows_v, irows_v, ub_v, ib_v,
             bias_v, out_v, sem):
    wid = lax.axis_index("s") * _NC + lax.axis_index("c")
    base = wid * _BPW

    pltpu.sync_copy(user_hbm.at[pl.ds(base, _BPW)], uidx_v)
    pltpu.sync_copy(item_hbm.at[pl.ds(base, _BPW)], iidx_v)
    pltpu.sync_copy(b_hbm, bias_v)

    cp_u = pltpu.async_copy(eu_hbm.at[uidx_v], urows_v, sem)
    cp_i = pltpu.async_copy(ei_hbm.at[iidx_v], irows_v, sem)
    cp_ub = pltpu.async_copy(ub_hbm.at[uidx_v], ub_v, sem)
    cp_ib = pltpu.async_copy(ib_hbm.at[iidx_v], ib_v, sem)
    cp_u.wait()
    cp_i.wait()
    cp_ub.wait()
    cp_ib.wait()

    bias = bias_v[...]
    lanes = lax.iota(jnp.int32, _L)

    def group(g, carry):
        sl = pl.ds(g * _L, _L)
        row = g * _L + lanes
        acc = bias + ub_v[sl] + ib_v[sl]
        for f in range(FACTORS):
            col = jnp.full((_L,), f, jnp.int32)
            u = plsc.load_gather(urows_v, [row, col])
            v = plsc.load_gather(irows_v, [row, col])
            acc = acc + u * v
        out_v[sl] = acc
        return carry

    lax.fori_loop(0, _GROUPS, group, 0)
    pltpu.sync_copy(out_v, out_hbm.at[pl.ds(base, _BPW)])


def kernel(user, item, embed_user, embed_item, u_bias, i_bias, bias_):
    mesh = plsc.VectorSubcoreMesh(core_axis_name="c", subcore_axis_name="s")
    fm = pl.kernel(
        _fm_body,
        out_type=jax.ShapeDtypeStruct((BATCH,), jnp.float32),
        mesh=mesh,
        compiler_params=pltpu.CompilerParams(
            needs_layout_passes=False, use_tc_tiling_on_sc=False),
        scratch_types=[
            pltpu.VMEM((_BPW,), jnp.int32),
            pltpu.VMEM((_BPW,), jnp.int32),
            pltpu.VMEM((_BPW, FACTORS), jnp.float32),
            pltpu.VMEM((_BPW, FACTORS), jnp.float32),
            pltpu.VMEM((_BPW,), jnp.float32),
            pltpu.VMEM((_BPW,), jnp.float32),
            pltpu.VMEM((_L,), jnp.float32),
            pltpu.VMEM((_BPW,), jnp.float32),
            pltpu.SemaphoreType.DMA,
        ],
    )
    return fm(user.astype(jnp.int32), item.astype(jnp.int32),
              embed_user, embed_item,
              u_bias.reshape(-1), i_bias.reshape(-1),
              jnp.broadcast_to(bias_, (_L,)))
